# baseline (speedup 1.0000x reference)
import functools

import jax
import jax.numpy as jnp
from jax.experimental import pallas as pl
from jax.experimental.pallas import tpu as pltpu

BN_EPS = 1e-5
KSIZE = 3
PAD = 1
VMEM_LIMIT_BYTES = 64 * 1024 * 1024


def _round_up(x, m):
    return (x + m - 1) // m * m


def _init_mask(mask_ref, *, w_pad, w_out):
    @pl.when(pl.program_id(0) == 0)
    def _():
        col = jax.lax.broadcasted_iota(jnp.int32, mask_ref.shape, 0) % w_pad
        mask_ref[...] = (col < w_out).astype(jnp.float32)


def _acc_stats(acc, mask, sum_ref, ssq_ref, *, init):
    yv = acc * mask
    s = jnp.sum(yv, axis=0, keepdims=True)
    q = jnp.sum(yv * acc, axis=0, keepdims=True)

    @pl.when(init)
    def _():
        sum_ref[...] = jnp.zeros_like(sum_ref)
        ssq_ref[...] = jnp.zeros_like(ssq_ref)

    sum_ref[0, :1, :] += s
    ssq_ref[0, :1, :] += q


def _conv_from_ref(slice_fn, w_ref, *, tap_offsets, base):
    acc = None
    for t, off in enumerate(tap_offsets):
        lhs = slice_fn(base + off)
        part = jnp.dot(lhs, w_ref[t], preferred_element_type=jnp.float32)
        acc = part if acc is None else acc + part
    return acc


def _conv_stats_kernel(x_ref, w_ref, y_ref, sum_ref, ssq_ref,
                       scratch_ref, mask_ref, *,
                       m_rows, h, w, w_pad, w_out, tap_offsets,
                       s_off, s_rows):
    _init_mask(mask_ref, w_pad=w_pad, w_out=w_out)

    @pl.when(pl.program_id(0) == 0)
    def _():
        scratch_ref[...] = jnp.zeros(scratch_ref.shape, scratch_ref.dtype)

    for b in range(y_ref.shape[0]):
        xt = jnp.transpose(x_ref[b], (1, 0)).astype(jnp.bfloat16)
        for hh in range(h):
            scratch_ref[pl.ds(s_off + hh * w_pad, w), :] = (
                xt[hh * w:(hh + 1) * w])
        acc = _conv_from_ref(lambda o: scratch_ref[pl.ds(o, m_rows), :],
                             w_ref, tap_offsets=tap_offsets,
                             base=s_off - (w_pad + 1))
        _acc_stats(acc, mask_ref[...], sum_ref, ssq_ref,
                   init=(b == 0) & (pl.program_id(0) == 0))
        y_ref[b] = acc.astype(y_ref.dtype)


def _bn_conv_stats_kernel(y1_ref, s_ref, q_ref, g_ref, b_ref, w_ref,
                          y2_ref, sum_ref, ssq_ref, scratch_ref, mask_ref, *,
                          m_rows, w_pad, w_out, tap_offsets, s_off, s_rows,
                          count):
    _init_mask(mask_ref, w_pad=w_pad, w_out=w_out)
    a, c = _fold_bn_rows(s_ref[0, :1, :], q_ref[0, :1, :], g_ref[...],
                         b_ref[...], count)

    @pl.when(pl.program_id(0) == 0)
    def _():
        cols = mask_ref.shape[1]
        scratch_ref[pl.ds(0, s_off), :] = jnp.zeros(
            (s_off, cols), scratch_ref.dtype)
        scratch_ref[pl.ds(s_off + m_rows, s_rows - s_off - m_rows), :] = (
            jnp.zeros((s_rows - s_off - m_rows, cols), scratch_ref.dtype))

    for b in range(y2_ref.shape[0]):
        z = jnp.maximum(y1_ref[b].astype(jnp.float32) * a + c,
                        0.0) * mask_ref[...]
        scratch_ref[pl.ds(s_off, m_rows), :] = z.astype(scratch_ref.dtype)
        acc = _conv_from_ref(lambda o: scratch_ref[pl.ds(o, m_rows), :],
                             w_ref, tap_offsets=tap_offsets,
                             base=s_off - (w_pad + 1))
        _acc_stats(acc, mask_ref[...], sum_ref, ssq_ref,
                   init=(b == 0) & (pl.program_id(0) == 0))
        y2_ref[b] = acc.astype(y2_ref.dtype)


def _bn_relu_t_kernel(y_ref, s_ref, q_ref, g_ref, b_ref, o_ref, *,
                      h_out, w_out, w_pad, count):
    a, c = _fold_bn_rows(s_ref[0, :1, :], q_ref[0, :1, :], g_ref[...],
                         b_ref[...], count)
    for b in range(o_ref.shape[0]):
        chunks = [y_ref[b, pl.ds(hh * w_pad, w_out), :]
                  for hh in range(h_out)]
        yc = jnp.concatenate(chunks, axis=0)
        z = jnp.maximum(yc.astype(jnp.float32) * a + c, 0.0)
        o_ref[b] = jnp.transpose(z, (1, 0))


def _fold_bn_rows(s_row, q_row, g_row, b_row, count):
    mean = s_row * (1.0 / count)
    var = jnp.maximum(q_row * (1.0 / count) - mean * mean, 0.0)
    a = g_row * jax.lax.rsqrt(var + BN_EPS)
    c = b_row - mean * a
    return a, c


def _weight_taps(weight):
    w = jnp.transpose(weight, (2, 3, 1, 0))
    k = weight.shape[-1]
    return w.reshape(k * k, weight.shape[1], weight.shape[0]).astype(
        jnp.bfloat16)


def kernel(x, l1_w, l1_b, l1_g, l1_beta, l2_w, l2_b, l2_g, l2_beta):
    del l1_b, l2_b
    n, cin, h, w = x.shape
    mid = l1_w.shape[0]
    cout = l2_w.shape[0]
    h_pad, w_pad = h + 2 * PAD, w + 2 * PAD
    h_out, w_out = h_pad - KSIZE + 1, w_pad - KSIZE + 1
    m_rows = h_out * w_pad
    p_in = _round_up(h_pad * w_pad + KSIZE - 1, 16)
    tap_offsets = tuple(kh * w_pad + kw
                        for kh in range(KSIZE) for kw in range(KSIZE))
    s_off = 80
    s_rows = _round_up(s_off - (w_pad + 1) + tap_offsets[-1] + m_rows, 16)
    count = n * h_out * w_out

    x3 = x.reshape(n, cin, h * w)
    w1 = _weight_taps(l1_w)
    w2 = _weight_taps(l2_w)

    conv_flops = 2 * n * m_rows * KSIZE * KSIZE * cin * mid
    ipb = 2 if n % 2 == 0 else 1
    grid = (n // ipb,)
    stats_specs = [
        pl.BlockSpec((1, 8, mid), lambda i: (0, 0, 0)),
        pl.BlockSpec((1, 8, mid), lambda i: (0, 0, 0)),
    ]

    y1, s1, q1 = pl.pallas_call(
        functools.partial(_conv_stats_kernel, m_rows=m_rows, h=h, w=w,
                          w_pad=w_pad, w_out=w_out, tap_offsets=tap_offsets,
                          s_off=s_off, s_rows=s_rows),
        out_shape=(
            jax.ShapeDtypeStruct((n, m_rows, mid), jnp.bfloat16),
            jax.ShapeDtypeStruct((1, 8, mid), jnp.float32),
            jax.ShapeDtypeStruct((1, 8, mid), jnp.float32),
        ),
        grid_spec=pltpu.PrefetchScalarGridSpec(
            num_scalar_prefetch=0,
            grid=grid,
            in_specs=[
                pl.BlockSpec((ipb, cin, h * w), lambda i: (i, 0, 0)),
                pl.BlockSpec((KSIZE * KSIZE, cin, mid), lambda i: (0, 0, 0)),
            ],
            out_specs=[pl.BlockSpec((ipb, m_rows, mid), lambda i: (i, 0, 0))]
            + stats_specs,
            scratch_shapes=[pltpu.VMEM((s_rows, cin), jnp.bfloat16),
                            pltpu.VMEM((m_rows, mid), jnp.float32)],
        ),
        compiler_params=pltpu.CompilerParams(
            dimension_semantics=("arbitrary",),
            vmem_limit_bytes=VMEM_LIMIT_BYTES,
        ),
        cost_estimate=pl.CostEstimate(
            flops=conv_flops, transcendentals=0,
            bytes_accessed=4 * n * h * w * cin + 2 * n * m_rows * mid),
    )(x3, w1)

    g1 = l1_g.reshape(1, mid)
    b1 = l1_beta.reshape(1, mid)
    y2, s2, q2 = pl.pallas_call(
        functools.partial(_bn_conv_stats_kernel, m_rows=m_rows, w_pad=w_pad,
                          w_out=w_out, tap_offsets=tap_offsets,
                          s_off=s_off, s_rows=s_rows, count=count),
        out_shape=(
            jax.ShapeDtypeStruct((n, m_rows, cout), jnp.bfloat16),
            jax.ShapeDtypeStruct((1, 8, cout), jnp.float32),
            jax.ShapeDtypeStruct((1, 8, cout), jnp.float32),
        ),
        grid_spec=pltpu.PrefetchScalarGridSpec(
            num_scalar_prefetch=0,
            grid=grid,
            in_specs=[
                pl.BlockSpec((ipb, m_rows, mid), lambda i: (i, 0, 0)),
                pl.BlockSpec((1, 8, mid), lambda i: (0, 0, 0)),
                pl.BlockSpec((1, 8, mid), lambda i: (0, 0, 0)),
                pl.BlockSpec((1, mid), lambda i: (0, 0)),
                pl.BlockSpec((1, mid), lambda i: (0, 0)),
                pl.BlockSpec((KSIZE * KSIZE, mid, cout), lambda i: (0, 0, 0)),
            ],
            out_specs=[pl.BlockSpec((ipb, m_rows, cout), lambda i: (i, 0, 0))]
            + stats_specs,
            scratch_shapes=[pltpu.VMEM((s_rows, mid), jnp.bfloat16),
                            pltpu.VMEM((m_rows, cout), jnp.float32)],
        ),
        compiler_params=pltpu.CompilerParams(
            dimension_semantics=("arbitrary",),
            vmem_limit_bytes=VMEM_LIMIT_BYTES,
        ),
        cost_estimate=pl.CostEstimate(
            flops=conv_flops, transcendentals=0,
            bytes_accessed=2 * (n * m_rows * mid + n * m_rows * cout)),
    )(y1, s1, q1, g1, b1, w2)

    g2 = l2_g.reshape(1, cout)
    b2 = l2_beta.reshape(1, cout)
    out_t = pl.pallas_call(
        functools.partial(_bn_relu_t_kernel, h_out=h_out, w_out=w_out,
                          w_pad=w_pad, count=count),
        out_shape=jax.ShapeDtypeStruct((n, cout, h_out * w_out), jnp.float32),
        grid_spec=pltpu.PrefetchScalarGridSpec(
            num_scalar_prefetch=0,
            grid=grid,
            in_specs=[
                pl.BlockSpec((ipb, m_rows, cout), lambda i: (i, 0, 0)),
                pl.BlockSpec((1, 8, cout), lambda i: (0, 0, 0)),
                pl.BlockSpec((1, 8, cout), lambda i: (0, 0, 0)),
                pl.BlockSpec((1, cout), lambda i: (0, 0)),
                pl.BlockSpec((1, cout), lambda i: (0, 0)),
            ],
            out_specs=pl.BlockSpec((ipb, cout, h_out * w_out),
                                   lambda i: (i, 0, 0)),
        ),
        compiler_params=pltpu.CompilerParams(
            dimension_semantics=("arbitrary",),
            vmem_limit_bytes=VMEM_LIMIT_BYTES,
        ),
        cost_estimate=pl.CostEstimate(
            flops=2 * n * m_rows * cout, transcendentals=0,
            bytes_accessed=6 * n * m_rows * cout),
    )(y2, s2, q2, g2, b2)

    return out_t.reshape(n, cout, h_out, w_out)


# speedup vs baseline: 1.3267x; 1.3267x over previous
import functools

import jax
import jax.numpy as jnp
from jax.experimental import pallas as pl
from jax.experimental.pallas import tpu as pltpu

BN_EPS = 1e-5
KSIZE = 3
PAD = 1
VMEM_LIMIT_BYTES = 64 * 1024 * 1024


def _round_up(x, m):
    return (x + m - 1) // m * m


def _init_mask(mask_ref, *, w_pad, w_out):
    @pl.when(pl.program_id(0) == 0)
    def _():
        col = jax.lax.broadcasted_iota(jnp.int32, mask_ref.shape, 0) % w_pad
        mask_ref[...] = (col < w_out).astype(jnp.float32)


def _acc_stats(acc, mask, sum_ref, ssq_ref, *, init):
    yv = acc * mask
    s = jnp.sum(yv, axis=0, keepdims=True)
    q = jnp.sum(yv * acc, axis=0, keepdims=True)

    @pl.when(init)
    def _():
        sum_ref[...] = jnp.zeros_like(sum_ref)
        ssq_ref[...] = jnp.zeros_like(ssq_ref)

    sum_ref[0, :1, :] += s
    ssq_ref[0, :1, :] += q


def _conv_from_ref(slice_fn, w_ref, *, tap_offsets, base):
    acc = None
    for t, off in enumerate(tap_offsets):
        lhs = slice_fn(base + off)
        part = jnp.dot(lhs, w_ref[t], preferred_element_type=jnp.float32)
        acc = part if acc is None else acc + part
    return acc


def _conv_stats_kernel(x_ref, w_ref, y_ref, sum_ref, ssq_ref,
                       scratch_ref, mask_ref, *,
                       m_rows, h, w, w_pad, w_out, tap_offsets,
                       s_off, s_rows):
    _init_mask(mask_ref, w_pad=w_pad, w_out=w_out)

    @pl.when(pl.program_id(0) == 0)
    def _():
        scratch_ref[...] = jnp.zeros(scratch_ref.shape, scratch_ref.dtype)

    for b in range(y_ref.shape[0]):
        xt = jnp.transpose(x_ref[b], (1, 0)).astype(jnp.bfloat16)
        for hh in range(h):
            scratch_ref[pl.ds(s_off + hh * w_pad, w), :] = (
                xt[hh * w:(hh + 1) * w])
        acc = _conv_from_ref(lambda o: scratch_ref[pl.ds(o, m_rows), :],
                             w_ref, tap_offsets=tap_offsets,
                             base=s_off - (w_pad + 1))
        _acc_stats(acc, mask_ref[...], sum_ref, ssq_ref,
                   init=(b == 0) & (pl.program_id(0) == 0))
        y_ref[b] = acc.astype(y_ref.dtype)


def _bn_conv_stats_kernel(y1_ref, s_ref, q_ref, g_ref, b_ref, w_ref,
                          y2_ref, sum_ref, ssq_ref, scratch_ref, mask_ref, *,
                          m_rows, w_pad, w_out, tap_offsets, s_off, s_rows,
                          count):
    _init_mask(mask_ref, w_pad=w_pad, w_out=w_out)
    a, c = _fold_bn_rows(s_ref[0, :1, :], q_ref[0, :1, :], g_ref[...],
                         b_ref[...], count)

    @pl.when(pl.program_id(0) == 0)
    def _():
        cols = mask_ref.shape[1]
        scratch_ref[pl.ds(0, s_off), :] = jnp.zeros(
            (s_off, cols), scratch_ref.dtype)
        scratch_ref[pl.ds(s_off + m_rows, s_rows - s_off - m_rows), :] = (
            jnp.zeros((s_rows - s_off - m_rows, cols), scratch_ref.dtype))

    for b in range(y2_ref.shape[0]):
        z = jnp.maximum(y1_ref[b].astype(jnp.float32) * a + c,
                        0.0) * mask_ref[...]
        scratch_ref[pl.ds(s_off, m_rows), :] = z.astype(scratch_ref.dtype)
        acc = _conv_from_ref(lambda o: scratch_ref[pl.ds(o, m_rows), :],
                             w_ref, tap_offsets=tap_offsets,
                             base=s_off - (w_pad + 1))
        _acc_stats(acc, mask_ref[...], sum_ref, ssq_ref,
                   init=(b == 0) & (pl.program_id(0) == 0))
        y2_ref[b] = acc.astype(y2_ref.dtype)


def _bn_relu_t_kernel(y_ref, s_ref, q_ref, g_ref, b_ref, o_ref, *,
                      h_out, w_out, w_pad, count):
    a, c = _fold_bn_rows(s_ref[0, :1, :], q_ref[0, :1, :], g_ref[...],
                         b_ref[...], count)
    for b in range(o_ref.shape[0]):
        chunks = [y_ref[b, pl.ds(hh * w_pad, w_out), :]
                  for hh in range(h_out)]
        yc = jnp.concatenate(chunks, axis=0)
        z = jnp.maximum(yc.astype(jnp.float32) * a + c, 0.0)
        o_ref[b] = jnp.transpose(z, (1, 0))


def _fold_bn_rows(s_row, q_row, g_row, b_row, count):
    mean = s_row * (1.0 / count)
    var = jnp.maximum(q_row * (1.0 / count) - mean * mean, 0.0)
    a = g_row * jax.lax.rsqrt(var + BN_EPS)
    c = b_row - mean * a
    return a, c


def _weight_taps(weight):
    w = jnp.transpose(weight, (2, 3, 1, 0))
    k = weight.shape[-1]
    return w.reshape(k * k, weight.shape[1], weight.shape[0]).astype(
        jnp.bfloat16)


def kernel(x, l1_w, l1_b, l1_g, l1_beta, l2_w, l2_b, l2_g, l2_beta):
    del l1_b, l2_b
    n, cin, h, w = x.shape
    mid = l1_w.shape[0]
    cout = l2_w.shape[0]
    h_pad, w_pad = h + 2 * PAD, w + 2 * PAD
    h_out, w_out = h_pad - KSIZE + 1, w_pad - KSIZE + 1
    m_rows = h_out * w_pad
    p_in = _round_up(h_pad * w_pad + KSIZE - 1, 16)
    tap_offsets = tuple(kh * w_pad + kw
                        for kh in range(KSIZE) for kw in range(KSIZE))
    s_off = 80
    s_rows = _round_up(s_off - (w_pad + 1) + tap_offsets[-1] + m_rows, 16)
    count = n * h_out * w_out

    x3 = x.reshape(n, cin, h * w)
    w1 = _weight_taps(l1_w)
    w2 = _weight_taps(l2_w)

    conv_flops = 2 * n * m_rows * KSIZE * KSIZE * cin * mid
    ipb = 1
    grid = (n // ipb,)
    stats_specs = [
        pl.BlockSpec((1, 8, mid), lambda i: (0, 0, 0)),
        pl.BlockSpec((1, 8, mid), lambda i: (0, 0, 0)),
    ]

    y1, s1, q1 = pl.pallas_call(
        functools.partial(_conv_stats_kernel, m_rows=m_rows, h=h, w=w,
                          w_pad=w_pad, w_out=w_out, tap_offsets=tap_offsets,
                          s_off=s_off, s_rows=s_rows),
        out_shape=(
            jax.ShapeDtypeStruct((n, m_rows, mid), jnp.bfloat16),
            jax.ShapeDtypeStruct((1, 8, mid), jnp.float32),
            jax.ShapeDtypeStruct((1, 8, mid), jnp.float32),
        ),
        grid_spec=pltpu.PrefetchScalarGridSpec(
            num_scalar_prefetch=0,
            grid=grid,
            in_specs=[
                pl.BlockSpec((ipb, cin, h * w), lambda i: (i, 0, 0)),
                pl.BlockSpec((KSIZE * KSIZE, cin, mid), lambda i: (0, 0, 0)),
            ],
            out_specs=[pl.BlockSpec((ipb, m_rows, mid), lambda i: (i, 0, 0))]
            + stats_specs,
            scratch_shapes=[pltpu.VMEM((s_rows, cin), jnp.bfloat16),
                            pltpu.VMEM((m_rows, mid), jnp.float32)],
        ),
        compiler_params=pltpu.CompilerParams(
            dimension_semantics=("arbitrary",),
            vmem_limit_bytes=VMEM_LIMIT_BYTES,
        ),
        cost_estimate=pl.CostEstimate(
            flops=conv_flops, transcendentals=0,
            bytes_accessed=4 * n * h * w * cin + 2 * n * m_rows * mid),
    )(x3, w1)

    g1 = l1_g.reshape(1, mid)
    b1 = l1_beta.reshape(1, mid)
    y2, s2, q2 = pl.pallas_call(
        functools.partial(_bn_conv_stats_kernel, m_rows=m_rows, w_pad=w_pad,
                          w_out=w_out, tap_offsets=tap_offsets,
                          s_off=s_off, s_rows=s_rows, count=count),
        out_shape=(
            jax.ShapeDtypeStruct((n, m_rows, cout), jnp.bfloat16),
            jax.ShapeDtypeStruct((1, 8, cout), jnp.float32),
            jax.ShapeDtypeStruct((1, 8, cout), jnp.float32),
        ),
        grid_spec=pltpu.PrefetchScalarGridSpec(
            num_scalar_prefetch=0,
            grid=grid,
            in_specs=[
                pl.BlockSpec((ipb, m_rows, mid), lambda i: (i, 0, 0)),
                pl.BlockSpec((1, 8, mid), lambda i: (0, 0, 0)),
                pl.BlockSpec((1, 8, mid), lambda i: (0, 0, 0)),
                pl.BlockSpec((1, mid), lambda i: (0, 0)),
                pl.BlockSpec((1, mid), lambda i: (0, 0)),
                pl.BlockSpec((KSIZE * KSIZE, mid, cout), lambda i: (0, 0, 0)),
            ],
            out_specs=[pl.BlockSpec((ipb, m_rows, cout), lambda i: (i, 0, 0))]
            + stats_specs,
            scratch_shapes=[pltpu.VMEM((s_rows, mid), jnp.bfloat16),
                            pltpu.VMEM((m_rows, cout), jnp.float32)],
        ),
        compiler_params=pltpu.CompilerParams(
            dimension_semantics=("arbitrary",),
            vmem_limit_bytes=VMEM_LIMIT_BYTES,
        ),
        cost_estimate=pl.CostEstimate(
            flops=conv_flops, transcendentals=0,
            bytes_accessed=2 * (n * m_rows * mid + n * m_rows * cout)),
    )(y1, s1, q1, g1, b1, w2)

    g2 = l2_g.reshape(1, cout)
    b2 = l2_beta.reshape(1, cout)
    out_t = pl.pallas_call(
        functools.partial(_bn_relu_t_kernel, h_out=h_out, w_out=w_out,
                          w_pad=w_pad, count=count),
        out_shape=jax.ShapeDtypeStruct((n, cout, h_out * w_out), jnp.float32),
        grid_spec=pltpu.PrefetchScalarGridSpec(
            num_scalar_prefetch=0,
            grid=grid,
            in_specs=[
                pl.BlockSpec((ipb, m_rows, cout), lambda i: (i, 0, 0)),
                pl.BlockSpec((1, 8, cout), lambda i: (0, 0, 0)),
                pl.BlockSpec((1, 8, cout), lambda i: (0, 0, 0)),
                pl.BlockSpec((1, cout), lambda i: (0, 0)),
                pl.BlockSpec((1, cout), lambda i: (0, 0)),
            ],
            out_specs=pl.BlockSpec((ipb, cout, h_out * w_out),
                                   lambda i: (i, 0, 0)),
        ),
        compiler_params=pltpu.CompilerParams(
            dimension_semantics=("arbitrary",),
            vmem_limit_bytes=VMEM_LIMIT_BYTES,
        ),
        cost_estimate=pl.CostEstimate(
            flops=2 * n * m_rows * cout, transcendentals=0,
            bytes_accessed=6 * n * m_rows * cout),
    )(y2, s2, q2, g2, b2)

    return out_t.reshape(n, cout, h_out, w_out)


# speedup vs baseline: 1.3494x; 1.0171x over previous
import functools

import jax
import jax.numpy as jnp
from jax.experimental import pallas as pl
from jax.experimental.pallas import tpu as pltpu

BN_EPS = 1e-5
KSIZE = 3
PAD = 1
VMEM_LIMIT_BYTES = 64 * 1024 * 1024


def _round_up(x, m):
    return (x + m - 1) // m * m


def _init_mask(mask_ref, *, w_pad, w_out):
    @pl.when(pl.program_id(0) == 0)
    def _():
        col = jax.lax.broadcasted_iota(jnp.int32, mask_ref.shape, 0) % w_pad
        mask_ref[...] = (col < w_out).astype(jnp.float32)


def _acc_stats(acc, mask, sum_ref, ssq_ref, *, init):
    yv = acc * mask
    s = jnp.sum(yv, axis=0, keepdims=True)
    q = jnp.sum(yv * acc, axis=0, keepdims=True)

    @pl.when(init)
    def _():
        sum_ref[...] = jnp.zeros_like(sum_ref)
        ssq_ref[...] = jnp.zeros_like(ssq_ref)

    sum_ref[0, :1, :] += s
    ssq_ref[0, :1, :] += q


def _conv_from_ref(slice_fn, w_ref, *, tap_offsets, base):
    acc = None
    for t, off in enumerate(tap_offsets):
        lhs = slice_fn(base + off)
        part = jnp.dot(lhs, w_ref[t], preferred_element_type=jnp.float32)
        acc = part if acc is None else acc + part
    return acc


def _conv_stats_kernel(x_ref, w_ref, y_ref, sum_ref, ssq_ref,
                       scratch_ref, mask_ref, *,
                       m_rows, h, w, w_pad, w_out, tap_offsets,
                       s_off, s_rows):
    _init_mask(mask_ref, w_pad=w_pad, w_out=w_out)

    @pl.when(pl.program_id(0) == 0)
    def _():
        scratch_ref[...] = jnp.zeros(scratch_ref.shape, scratch_ref.dtype)

    for b in range(y_ref.shape[0]):
        xt = jnp.transpose(x_ref[b], (1, 0)).astype(jnp.bfloat16)
        for hh in range(h):
            scratch_ref[pl.ds(s_off + hh * w_pad, w), :] = (
                xt[hh * w:(hh + 1) * w])
        acc = _conv_from_ref(lambda o: scratch_ref[pl.ds(o, m_rows), :],
                             w_ref, tap_offsets=tap_offsets,
                             base=s_off - (w_pad + 1))
        y_ref[b] = acc.astype(y_ref.dtype)
        _acc_stats(acc, mask_ref[...], sum_ref, ssq_ref,
                   init=(b == 0) & (pl.program_id(0) == 0))


def _bn_conv_stats_kernel(y1_ref, s_ref, q_ref, g_ref, b_ref, w_ref,
                          y2_ref, sum_ref, ssq_ref, scratch_ref, mask_ref, *,
                          m_rows, w_pad, w_out, tap_offsets, s_off, s_rows,
                          count):
    _init_mask(mask_ref, w_pad=w_pad, w_out=w_out)
    a, c = _fold_bn_rows(s_ref[0, :1, :], q_ref[0, :1, :], g_ref[...],
                         b_ref[...], count)

    @pl.when(pl.program_id(0) == 0)
    def _():
        cols = mask_ref.shape[1]
        scratch_ref[pl.ds(0, s_off), :] = jnp.zeros(
            (s_off, cols), scratch_ref.dtype)
        scratch_ref[pl.ds(s_off + m_rows, s_rows - s_off - m_rows), :] = (
            jnp.zeros((s_rows - s_off - m_rows, cols), scratch_ref.dtype))

    for b in range(y2_ref.shape[0]):
        z = jnp.maximum(y1_ref[b].astype(jnp.float32) * a + c,
                        0.0) * mask_ref[...]
        scratch_ref[pl.ds(s_off, m_rows), :] = z.astype(scratch_ref.dtype)
        acc = _conv_from_ref(lambda o: scratch_ref[pl.ds(o, m_rows), :],
                             w_ref, tap_offsets=tap_offsets,
                             base=s_off - (w_pad + 1))
        y2_ref[b] = acc.astype(y2_ref.dtype)
        _acc_stats(acc, mask_ref[...], sum_ref, ssq_ref,
                   init=(b == 0) & (pl.program_id(0) == 0))


def _bn_relu_t_kernel(y_ref, s_ref, q_ref, g_ref, b_ref, o_ref, *,
                      h_out, w_out, w_pad, count):
    a, c = _fold_bn_rows(s_ref[0, :1, :], q_ref[0, :1, :], g_ref[...],
                         b_ref[...], count)
    for b in range(o_ref.shape[0]):
        chunks = [y_ref[b, pl.ds(hh * w_pad, w_out), :]
                  for hh in range(h_out)]
        yc = jnp.concatenate(chunks, axis=0)
        z = jnp.maximum(yc.astype(jnp.float32) * a + c, 0.0)
        o_ref[b] = jnp.transpose(z, (1, 0))


def _fold_bn_rows(s_row, q_row, g_row, b_row, count):
    mean = s_row * (1.0 / count)
    var = jnp.maximum(q_row * (1.0 / count) - mean * mean, 0.0)
    a = g_row * jax.lax.rsqrt(var + BN_EPS)
    c = b_row - mean * a
    return a, c


def _weight_taps(weight):
    w = jnp.transpose(weight, (2, 3, 1, 0))
    k = weight.shape[-1]
    return w.reshape(k * k, weight.shape[1], weight.shape[0]).astype(
        jnp.bfloat16)


def kernel(x, l1_w, l1_b, l1_g, l1_beta, l2_w, l2_b, l2_g, l2_beta):
    del l1_b, l2_b
    n, cin, h, w = x.shape
    mid = l1_w.shape[0]
    cout = l2_w.shape[0]
    h_pad, w_pad = h + 2 * PAD, w + 2 * PAD
    h_out, w_out = h_pad - KSIZE + 1, w_pad - KSIZE + 1
    m_rows = h_out * w_pad
    p_in = _round_up(h_pad * w_pad + KSIZE - 1, 16)
    tap_offsets = tuple(kh * w_pad + kw
                        for kh in range(KSIZE) for kw in range(KSIZE))
    s_off = 80
    s_rows = _round_up(s_off - (w_pad + 1) + tap_offsets[-1] + m_rows, 16)
    count = n * h_out * w_out

    x3 = x.reshape(n, cin, h * w)
    w1 = _weight_taps(l1_w)
    w2 = _weight_taps(l2_w)

    conv_flops = 2 * n * m_rows * KSIZE * KSIZE * cin * mid
    ipb = 1
    grid = (n // ipb,)
    stats_specs = [
        pl.BlockSpec((1, 8, mid), lambda i: (0, 0, 0)),
        pl.BlockSpec((1, 8, mid), lambda i: (0, 0, 0)),
    ]

    y1, s1, q1 = pl.pallas_call(
        functools.partial(_conv_stats_kernel, m_rows=m_rows, h=h, w=w,
                          w_pad=w_pad, w_out=w_out, tap_offsets=tap_offsets,
                          s_off=s_off, s_rows=s_rows),
        out_shape=(
            jax.ShapeDtypeStruct((n, m_rows, mid), jnp.bfloat16),
            jax.ShapeDtypeStruct((1, 8, mid), jnp.float32),
            jax.ShapeDtypeStruct((1, 8, mid), jnp.float32),
        ),
        grid_spec=pltpu.PrefetchScalarGridSpec(
            num_scalar_prefetch=0,
            grid=grid,
            in_specs=[
                pl.BlockSpec((ipb, cin, h * w), lambda i: (i, 0, 0)),
                pl.BlockSpec((KSIZE * KSIZE, cin, mid), lambda i: (0, 0, 0)),
            ],
            out_specs=[pl.BlockSpec((ipb, m_rows, mid), lambda i: (i, 0, 0))]
            + stats_specs,
            scratch_shapes=[pltpu.VMEM((s_rows, cin), jnp.bfloat16),
                            pltpu.VMEM((m_rows, mid), jnp.float32)],
        ),
        compiler_params=pltpu.CompilerParams(
            dimension_semantics=("arbitrary",),
            vmem_limit_bytes=VMEM_LIMIT_BYTES,
        ),
        cost_estimate=pl.CostEstimate(
            flops=conv_flops, transcendentals=0,
            bytes_accessed=4 * n * h * w * cin + 2 * n * m_rows * mid),
    )(x3, w1)

    g1 = l1_g.reshape(1, mid)
    b1 = l1_beta.reshape(1, mid)
    y2, s2, q2 = pl.pallas_call(
        functools.partial(_bn_conv_stats_kernel, m_rows=m_rows, w_pad=w_pad,
                          w_out=w_out, tap_offsets=tap_offsets,
                          s_off=s_off, s_rows=s_rows, count=count),
        out_shape=(
            jax.ShapeDtypeStruct((n, m_rows, cout), jnp.bfloat16),
            jax.ShapeDtypeStruct((1, 8, cout), jnp.float32),
            jax.ShapeDtypeStruct((1, 8, cout), jnp.float32),
        ),
        grid_spec=pltpu.PrefetchScalarGridSpec(
            num_scalar_prefetch=0,
            grid=grid,
            in_specs=[
                pl.BlockSpec((ipb, m_rows, mid), lambda i: (i, 0, 0)),
                pl.BlockSpec((1, 8, mid), lambda i: (0, 0, 0)),
                pl.BlockSpec((1, 8, mid), lambda i: (0, 0, 0)),
                pl.BlockSpec((1, mid), lambda i: (0, 0)),
                pl.BlockSpec((1, mid), lambda i: (0, 0)),
                pl.BlockSpec((KSIZE * KSIZE, mid, cout), lambda i: (0, 0, 0)),
            ],
            out_specs=[pl.BlockSpec((ipb, m_rows, cout), lambda i: (i, 0, 0))]
            + stats_specs,
            scratch_shapes=[pltpu.VMEM((s_rows, mid), jnp.bfloat16),
                            pltpu.VMEM((m_rows, cout), jnp.float32)],
        ),
        compiler_params=pltpu.CompilerParams(
            dimension_semantics=("arbitrary",),
            vmem_limit_bytes=VMEM_LIMIT_BYTES,
        ),
        cost_estimate=pl.CostEstimate(
            flops=conv_flops, transcendentals=0,
            bytes_accessed=2 * (n * m_rows * mid + n * m_rows * cout)),
    )(y1, s1, q1, g1, b1, w2)

    g2 = l2_g.reshape(1, cout)
    b2 = l2_beta.reshape(1, cout)
    out_t = pl.pallas_call(
        functools.partial(_bn_relu_t_kernel, h_out=h_out, w_out=w_out,
                          w_pad=w_pad, count=count),
        out_shape=jax.ShapeDtypeStruct((n, cout, h_out * w_out), jnp.float32),
        grid_spec=pltpu.PrefetchScalarGridSpec(
            num_scalar_prefetch=0,
            grid=grid,
            in_specs=[
                pl.BlockSpec((ipb, m_rows, cout), lambda i: (i, 0, 0)),
                pl.BlockSpec((1, 8, cout), lambda i: (0, 0, 0)),
                pl.BlockSpec((1, 8, cout), lambda i: (0, 0, 0)),
                pl.BlockSpec((1, cout), lambda i: (0, 0)),
                pl.BlockSpec((1, cout), lambda i: (0, 0)),
            ],
            out_specs=pl.BlockSpec((ipb, cout, h_out * w_out),
                                   lambda i: (i, 0, 0)),
        ),
        compiler_params=pltpu.CompilerParams(
            dimension_semantics=("arbitrary",),
            vmem_limit_bytes=VMEM_LIMIT_BYTES,
        ),
        cost_estimate=pl.CostEstimate(
            flops=2 * n * m_rows * cout, transcendentals=0,
            bytes_accessed=6 * n * m_rows * cout),
    )(y2, s2, q2, g2, b2)

    return out_t.reshape(n, cout, h_out, w_out)
